# baseline (speedup 1.0000x reference)
import jax
import jax.numpy as jnp
from jax import lax
from jax.experimental import pallas as pl
from jax.experimental.pallas import tpu as pltpu
from jax.experimental.pallas import tpu_sc as plsc

_ROWS = 16384
_COLS = 128
_NC = 2
_NS = 16
_NW = _NC * _NS
_STRIPE = _ROWS // _NW


def _sc_body(x_hbm, o_hbm, buf, in_sem, out_sem):
    wid = lax.axis_index("s") * _NC + lax.axis_index("c")
    base = wid * _STRIPE
    half = _STRIPE // 2
    cp_in0 = pltpu.make_async_copy(
        x_hbm.at[pl.ds(base, half), :], buf.at[0], in_sem.at[0]
    )
    cp_in1 = pltpu.make_async_copy(
        x_hbm.at[pl.ds(base + half, half), :], buf.at[1], in_sem.at[1]
    )
    cp_out0 = pltpu.make_async_copy(
        buf.at[0], o_hbm.at[pl.ds(base, half), :], out_sem.at[0]
    )
    cp_out1 = pltpu.make_async_copy(
        buf.at[1], o_hbm.at[pl.ds(base + half, half), :], out_sem.at[1]
    )
    cp_in0.start()
    cp_in1.start()
    cp_in0.wait()
    cp_out0.start()
    cp_in1.wait()
    cp_out1.start()
    cp_out0.wait()
    cp_out1.wait()


def kernel(x):
    mesh = plsc.VectorSubcoreMesh(
        core_axis_name="c", subcore_axis_name="s", num_cores=_NC, num_subcores=_NS
    )
    k = pl.kernel(
        _sc_body,
        out_type=jax.ShapeDtypeStruct((_ROWS, _COLS), jnp.float32),
        mesh=mesh,
        scratch_types=[
            pltpu.VMEM((2, _STRIPE // 2, _COLS), jnp.float32),
            pltpu.SemaphoreType.DMA((2,)),
            pltpu.SemaphoreType.DMA((2,)),
        ],
    )
    return k(x)


# speedup vs baseline: 4.1807x; 4.1807x over previous
import jax
import jax.numpy as jnp
from jax.experimental import pallas as pl
from jax.experimental.pallas import tpu as pltpu

_ROWS = 16384
_COLS = 128
_CHUNK = 4096
_NCHUNKS = _ROWS // _CHUNK


def _copy_body(x_hbm, o_hbm, buf, in_sem, out_sem):
    def in_cp(i):
        return pltpu.make_async_copy(
            x_hbm.at[pl.ds(i * _CHUNK, _CHUNK), :], buf.at[i], in_sem.at[i]
        )

    def out_cp(i):
        return pltpu.make_async_copy(
            buf.at[i], o_hbm.at[pl.ds(i * _CHUNK, _CHUNK), :], out_sem.at[i]
        )

    for i in range(_NCHUNKS):
        in_cp(i).start()
    for i in range(_NCHUNKS):
        in_cp(i).wait()
        out_cp(i).start()
    for i in range(_NCHUNKS):
        out_cp(i).wait()


def kernel(x):
    return pl.pallas_call(
        _copy_body,
        in_specs=[pl.BlockSpec(memory_space=pl.ANY)],
        out_specs=pl.BlockSpec(memory_space=pl.ANY),
        scratch_shapes=[
            pltpu.VMEM((_NCHUNKS, _CHUNK, _COLS), jnp.float32),
            pltpu.SemaphoreType.DMA((_NCHUNKS,)),
            pltpu.SemaphoreType.DMA((_NCHUNKS,)),
        ],
        out_shape=jax.ShapeDtypeStruct(x.shape, x.dtype),
    )(x)


# speedup vs baseline: 4.3560x; 1.0419x over previous
import jax
import jax.numpy as jnp
from jax.experimental import pallas as pl
from jax.experimental.pallas import tpu as pltpu

_ROWS = 16384
_COLS = 128
_CHUNK_ROWS = (1024, 2048, 5120, 5120, 2048, 1024)
_OFFSETS = tuple(sum(_CHUNK_ROWS[:i]) for i in range(len(_CHUNK_ROWS)))
_NCHUNKS = len(_CHUNK_ROWS)


def _copy_body(x_hbm, o_hbm, buf, in_sem, out_sem):
    def in_cp(i):
        off, n = _OFFSETS[i], _CHUNK_ROWS[i]
        return pltpu.make_async_copy(
            x_hbm.at[pl.ds(off, n), :], buf.at[pl.ds(off, n), :], in_sem.at[i]
        )

    def out_cp(i):
        off, n = _OFFSETS[i], _CHUNK_ROWS[i]
        return pltpu.make_async_copy(
            buf.at[pl.ds(off, n), :], o_hbm.at[pl.ds(off, n), :], out_sem.at[i]
        )

    for i in range(_NCHUNKS):
        in_cp(i).start()
    for i in range(_NCHUNKS):
        in_cp(i).wait()
        out_cp(i).start()
    for i in range(_NCHUNKS):
        out_cp(i).wait()


def kernel(x):
    return pl.pallas_call(
        _copy_body,
        in_specs=[pl.BlockSpec(memory_space=pl.ANY)],
        out_specs=pl.BlockSpec(memory_space=pl.ANY),
        scratch_shapes=[
            pltpu.VMEM((_ROWS, _COLS), jnp.float32),
            pltpu.SemaphoreType.DMA((_NCHUNKS,)),
            pltpu.SemaphoreType.DMA((_NCHUNKS,)),
        ],
        out_shape=jax.ShapeDtypeStruct(x.shape, x.dtype),
    )(x)
